# baseline (speedup 1.0000x reference)
import jax
import jax.numpy as jnp
from jax import lax
from jax.experimental import pallas as pl
from jax.experimental.pallas import tpu as pltpu, tpu_sc as plsc

VOCAB = 1000000
EMBED_DIM = 64
BATCH = 16384
FIELDS = 26

NC = 2
NS = 16
NW = NC * NS

ROWS_PER_W = BATCH // NW
NI = 16
CHUNKS_PER_W = ROWS_PER_W // NI
NBUF = 4
GROUPS = CHUNKS_PER_W // NBUF


def _embed_kernel(idx_hbm, table_hbm, out_hbm, idx_v, bufs, gsems, wsems):
    wid = lax.axis_index("s") * NC + lax.axis_index("c")
    row0 = wid * ROWS_PER_W
    pltpu.sync_copy(idx_hbm.at[pl.ds(row0, ROWS_PER_W)], idx_v)

    def group(g, carry):
        gdescs = []
        for b in range(NBUF):
            k = g * NBUF + b
            for r in range(NI):
                d = pltpu.async_copy(
                    table_hbm.at[idx_v.at[k * NI + r]],
                    bufs.at[b].at[r], gsems.at[b])
                gdescs.append(d)
        wdescs = []
        for b in range(NBUF):
            k = g * NBUF + b
            for r in range(NI):
                gdescs[b * NI + r].wait()
            d = pltpu.async_copy(
                bufs.at[b], out_hbm.at[pl.ds(row0 + k * NI, NI)],
                wsems.at[b])
            wdescs.append(d)
        for b in range(NBUF):
            wdescs[b].wait()
        return carry

    lax.fori_loop(0, GROUPS, group, 0)


def kernel(embed_input, weight):
    mesh = plsc.VectorSubcoreMesh(core_axis_name="c", subcore_axis_name="s")
    return pl.kernel(
        _embed_kernel,
        out_type=jax.ShapeDtypeStruct((BATCH, FIELDS, EMBED_DIM), jnp.float32),
        mesh=mesh,
        compiler_params=pltpu.CompilerParams(use_tc_tiling_on_sc=False),
        scratch_types=[
            pltpu.VMEM((ROWS_PER_W, FIELDS), jnp.int32),
            pltpu.VMEM((NBUF, NI, FIELDS, EMBED_DIM), jnp.float32),
            pltpu.SemaphoreType.DMA((NBUF,)),
            pltpu.SemaphoreType.DMA((NBUF,)),
        ],
    )(embed_input, weight)
